# baseline (speedup 1.0000x reference)
import numpy as np
import jax
import jax.numpy as jnp
from jax import lax
from jax.experimental import pallas as pl
from jax.experimental.pallas import tpu as pltpu
from jax.experimental.pallas import tpu_sc as plsc

B = 16
S = 64
K = 3
F = 384
EDIM = 32
PDIM = 32
TAGS = 7
OUTW = 2 * F + EDIM + PDIM
TAIL0 = 2 * F

_base = np.arange(1, S + 1)
_emo = np.repeat(_base, S)
_cau = np.tile(_base, S)
_rel = _cau - _emo
_msk = np.abs(_rel) <= K
I_P = (_emo[_msk] - 1).astype(np.int32)
J_P = (_cau[_msk] - 1).astype(np.int32)
R_P = (_rel[_msk] + K).astype(np.int32)
NPAIR = int(I_P.shape[0])
EMO_CAU = np.stack([_emo[_msk], _cau[_msk]], axis=1).astype(np.int32)

_u = np.arange(2 * K + 1)
_counts = (S - np.abs(_u - K)).astype(np.float64)
COEFF = (np.exp(-((_u[:, None] - _u[None, :]) ** 2).astype(np.float64))
         * _counts[None, :])

SEL_E = np.zeros((NPAIR, S), np.float32)
SEL_E[np.arange(NPAIR), I_P] = 1.0
_selr = np.zeros((NPAIR, TAGS), np.float64)
_selr[np.arange(NPAIR), R_P] = 1.0
SELR_COEFF = (_selr @ COEFF).astype(np.float32)

NW = 32
PPW = 14
USLOT = 4
CSLOT = 10


def _isent(p):
    lo = jnp.where(p < 4, 0, jnp.where(p < 9, 1, 2))
    hi = jnp.where(p < 427, 61, jnp.where(p < 432, 62, 63))
    return jnp.where(p < 15, lo, jnp.where(p < 421, (p - 15) // 7 + 3, hi))


def _gstart(i):
    lo = jnp.where(i == 0, 0, jnp.where(i == 1, 4, 9))
    hi = jnp.where(i == 62, 427, 432)
    return jnp.where(i < 3, lo, jnp.where(i < 62, 15 + 7 * (i - 3), hi))


def _jsent(p, i):
    return jnp.maximum(0, i - K) + (p - _gstart(i))


def _sc_body(he, hc, out, gidxuv, cidxv, ubuf, cbuf, semu, semc):
    cid = lax.axis_index("c")
    sid = lax.axis_index("s")
    wid = sid * 2 + cid
    s0 = jnp.minimum(wid * PPW, NPAIR - PPW)
    i0 = _isent(s0)
    jmin = jnp.maximum(0, i0 - K)

    bvec = jnp.arange(16, dtype=jnp.int32) * S
    for m in range(USLOT):
        gidxuv[pl.ds(m * B, B)] = bvec + jnp.minimum(i0 + m, S - 1)
    for m in range(CSLOT):
        cidxv[m // 5, pl.ds((m % 5) * B, B)] = bvec + jnp.minimum(
            jmin + m, S - 1)

    gu = pltpu.async_copy(he.at[gidxuv], ubuf, semu)
    gc0 = pltpu.async_copy(hc.at[cidxv.at[0]], cbuf.at[pl.ds(0, 5 * B)], semc)
    gc1 = pltpu.async_copy(hc.at[cidxv.at[1]], cbuf.at[pl.ds(5 * B, 5 * B)],
                           semc)

    ms, mcs = [], []
    for k in range(PPW):
        i = _isent(s0 + k)
        ms.append(i - i0)
        mcs.append(_jsent(s0 + k, i) - jmin)

    gu.wait()
    hw = []
    for k in range(PPW):
        hw.append(pltpu.async_copy(
            ubuf.at[pl.ds(ms[k] * B, B)],
            out.at[s0 + k, :, pl.ds(0, F)], semu))
    gc0.wait()
    gc1.wait()
    for k in range(PPW):
        hw.append(pltpu.async_copy(
            cbuf.at[pl.ds(mcs[k] * B, B)],
            out.at[s0 + k, :, pl.ds(F, F)], semc))
    for d in hw:
        d.wait()


NPB = NPAIR // 4


def _tc_tail_body(pe_ref, pos_ref, etab_ref, sele_ref, selrc_ref, ec_ref,
                  big_ref, out_ref, ec_out_ref):
    pe = pe_ref[...]
    am = jnp.argmax(pe, axis=-1)
    onehot = (lax.broadcasted_iota(jnp.int32, (B, S, TAGS), 2)
              == am[:, :, None]).astype(jnp.float32)
    emo_all = lax.dot_general(onehot, etab_ref[...], (((2,), (0,)), ((), ())),
                              precision=lax.Precision.HIGHEST)
    step = pl.program_id(0)
    sele_blk = sele_ref[pl.ds(step * NPB, NPB), :]
    selrc_blk = selrc_ref[pl.ds(step * NPB, NPB), :]
    emo_pairs = lax.dot_general(sele_blk, emo_all,
                                (((1,), (1,)), ((), ())),
                                precision=lax.Precision.HIGHEST)
    rel = lax.dot_general(selrc_blk, pos_ref[...],
                          (((1,), (0,)), ((), ())),
                          precision=lax.Precision.HIGHEST)
    rel_pairs = jnp.broadcast_to(rel[:, None, :], (NPB, B, PDIM))
    out_ref[:, :, 0:EDIM] = emo_pairs
    out_ref[:, :, EDIM:EDIM + PDIM] = rel_pairs
    ec_out_ref[...] = ec_ref[...]


def kernel(doc_sents_he, doc_sents_hc, pred_emo, pos_emb_weight,
           emo_emb_weight):
    he2 = doc_sents_he.reshape(B * S, F)
    hc2 = doc_sents_hc.reshape(B * S, F)

    mesh = plsc.VectorSubcoreMesh(core_axis_name="c", subcore_axis_name="s")
    scfn = pl.kernel(
        _sc_body,
        out_type=jax.ShapeDtypeStruct((NPAIR, B, OUTW), jnp.float32),
        mesh=mesh,
        scratch_types=[
            pltpu.VMEM((USLOT * B,), jnp.int32),
            pltpu.VMEM((2, 5 * B), jnp.int32),
            pltpu.VMEM((USLOT * B, F), jnp.float32),
            pltpu.VMEM((CSLOT * B, F), jnp.float32),
            pltpu.SemaphoreType.DMA,
            pltpu.SemaphoreType.DMA,
        ],
        compiler_params=pltpu.CompilerParams(needs_layout_passes=False),
    )
    big = scfn(he2, hc2)

    big, emo_cau = pl.pallas_call(
        _tc_tail_body,
        grid=(4,),
        in_specs=[
            pl.BlockSpec((B, S, TAGS), lambda i: (0, 0, 0)),
            pl.BlockSpec((TAGS, PDIM), lambda i: (0, 0)),
            pl.BlockSpec((TAGS, EDIM), lambda i: (0, 0)),
            pl.BlockSpec((NPAIR, S), lambda i: (0, 0)),
            pl.BlockSpec((NPAIR, TAGS), lambda i: (0, 0)),
            pl.BlockSpec((NPAIR, 2), lambda i: (0, 0)),
            pl.BlockSpec(memory_space=pl.ANY),
        ],
        out_specs=[
            pl.BlockSpec((NPB, B, 128), lambda i: (i, 0, TAIL0 // 128)),
            pl.BlockSpec((NPAIR, 2), lambda i: (0, 0)),
        ],
        out_shape=[
            jax.ShapeDtypeStruct((NPAIR, B, OUTW), jnp.float32),
            jax.ShapeDtypeStruct((NPAIR, 2), jnp.int32),
        ],
        input_output_aliases={6: 0},
    )(pred_emo, pos_emb_weight, emo_emb_weight, jnp.asarray(SEL_E),
      jnp.asarray(SELR_COEFF), jnp.asarray(EMO_CAU), big)

    couples = jnp.transpose(big, (1, 0, 2))
    return (couples, emo_cau)


# speedup vs baseline: 1.1013x; 1.1013x over previous
import numpy as np
import jax
import jax.numpy as jnp
from jax import lax
from jax.experimental import pallas as pl
from jax.experimental.pallas import tpu as pltpu
from jax.experimental.pallas import tpu_sc as plsc

B = 16
S = 64
K = 3
F = 384
EDIM = 32
PDIM = 32
TAGS = 7
OUTW = 2 * F + EDIM + PDIM
TAIL0 = 2 * F

_base = np.arange(1, S + 1)
_emo = np.repeat(_base, S)
_cau = np.tile(_base, S)
_rel = _cau - _emo
_msk = np.abs(_rel) <= K
I_P = (_emo[_msk] - 1).astype(np.int32)
J_P = (_cau[_msk] - 1).astype(np.int32)
R_P = (_rel[_msk] + K).astype(np.int32)
NPAIR = int(I_P.shape[0])
EMO_CAU = np.stack([_emo[_msk], _cau[_msk]], axis=1).astype(np.int32)

_u = np.arange(2 * K + 1)
_counts = (S - np.abs(_u - K)).astype(np.float64)
COEFF = (np.exp(-((_u[:, None] - _u[None, :]) ** 2).astype(np.float64))
         * _counts[None, :])

SEL_E = np.zeros((NPAIR, S), np.float32)
SEL_E[np.arange(NPAIR), I_P] = 1.0
_selr = np.zeros((NPAIR, TAGS), np.float64)
_selr[np.arange(NPAIR), R_P] = 1.0
SELR_COEFF = (_selr @ COEFF).astype(np.float32)

NW = 32
PPW = 14
USLOT = 4
CSLOT = 10


def _isent(p):
    lo = jnp.where(p < 4, 0, jnp.where(p < 9, 1, 2))
    hi = jnp.where(p < 427, 61, jnp.where(p < 432, 62, 63))
    return jnp.where(p < 15, lo, jnp.where(p < 421, (p - 15) // 7 + 3, hi))


def _gstart(i):
    lo = jnp.where(i == 0, 0, jnp.where(i == 1, 4, 9))
    hi = jnp.where(i == 62, 427, 432)
    return jnp.where(i < 3, lo, jnp.where(i < 62, 15 + 7 * (i - 3), hi))


def _jsent(p, i):
    return jnp.maximum(0, i - K) + (p - _gstart(i))


def _sc_body(he, hc, out, gidxuv, cidxv, ubuf, cbuf, semu, semc):
    cid = lax.axis_index("c")
    sid = lax.axis_index("s")
    wid = sid * 2 + cid
    s0 = jnp.minimum(wid * PPW, NPAIR - PPW)
    i0 = _isent(s0)
    jmin = jnp.maximum(0, i0 - K)

    bvec = jnp.arange(16, dtype=jnp.int32) * S
    for m in range(USLOT):
        gidxuv[pl.ds(m * B, B)] = bvec + jnp.minimum(i0 + m, S - 1)
    for m in range(CSLOT):
        cidxv[m // 5, pl.ds((m % 5) * B, B)] = bvec + jnp.minimum(
            jmin + m, S - 1)

    gu = pltpu.async_copy(he.at[gidxuv], ubuf, semu)
    gc0 = pltpu.async_copy(hc.at[cidxv.at[0]], cbuf.at[pl.ds(0, 5 * B)], semc)
    gc1 = pltpu.async_copy(hc.at[cidxv.at[1]], cbuf.at[pl.ds(5 * B, 5 * B)],
                           semc)

    ms, mcs = [], []
    for k in range(PPW):
        i = _isent(s0 + k)
        ms.append(i - i0)
        mcs.append(_jsent(s0 + k, i) - jmin)

    gu.wait()
    hw = []
    for k in range(PPW):
        hw.append(pltpu.async_copy(
            ubuf.at[pl.ds(ms[k] * B, B)],
            out.at[s0 + k, :, pl.ds(0, F)], semu))
    gc0.wait()
    gc1.wait()
    for k in range(PPW):
        hw.append(pltpu.async_copy(
            cbuf.at[pl.ds(mcs[k] * B, B)],
            out.at[s0 + k, :, pl.ds(F, F)], semc))
    for d in hw:
        d.wait()


def _tc_tail_body(pe_ref, pos_ref, etab_ref, sele_ref, selrc_ref, big_ref,
                  out_ref):
    pe = pe_ref[...]
    am = jnp.argmax(pe, axis=-1)
    onehot = (lax.broadcasted_iota(jnp.int32, (B, S, TAGS), 2)
              == am[:, :, None]).astype(jnp.float32)
    emo_all = lax.dot_general(onehot, etab_ref[...], (((2,), (0,)), ((), ())),
                              precision=lax.Precision.HIGHEST)
    emo_pairs = lax.dot_general(sele_ref[...], emo_all,
                                (((1,), (1,)), ((), ())),
                                precision=lax.Precision.HIGHEST)
    rel = lax.dot_general(selrc_ref[...], pos_ref[...],
                          (((1,), (0,)), ((), ())),
                          precision=lax.Precision.HIGHEST)
    rel_pairs = jnp.broadcast_to(rel[:, None, :], (NPAIR, B, PDIM))
    out_ref[:, :, 0:EDIM] = emo_pairs
    out_ref[:, :, EDIM:EDIM + PDIM] = rel_pairs


def kernel(doc_sents_he, doc_sents_hc, pred_emo, pos_emb_weight,
           emo_emb_weight):
    he2 = doc_sents_he.reshape(B * S, F)
    hc2 = doc_sents_hc.reshape(B * S, F)

    mesh = plsc.VectorSubcoreMesh(core_axis_name="c", subcore_axis_name="s")
    scfn = pl.kernel(
        _sc_body,
        out_type=jax.ShapeDtypeStruct((NPAIR, B, OUTW), jnp.float32),
        mesh=mesh,
        scratch_types=[
            pltpu.VMEM((USLOT * B,), jnp.int32),
            pltpu.VMEM((2, 5 * B), jnp.int32),
            pltpu.VMEM((USLOT * B, F), jnp.float32),
            pltpu.VMEM((CSLOT * B, F), jnp.float32),
            pltpu.SemaphoreType.DMA,
            pltpu.SemaphoreType.DMA,
        ],
        compiler_params=pltpu.CompilerParams(needs_layout_passes=False),
    )
    big = scfn(he2, hc2)

    big = pl.pallas_call(
        _tc_tail_body,
        grid=(1,),
        in_specs=[
            pl.BlockSpec((B, S, TAGS), lambda i: (0, 0, 0)),
            pl.BlockSpec((TAGS, PDIM), lambda i: (0, 0)),
            pl.BlockSpec((TAGS, EDIM), lambda i: (0, 0)),
            pl.BlockSpec((NPAIR, S), lambda i: (0, 0)),
            pl.BlockSpec((NPAIR, TAGS), lambda i: (0, 0)),
            pl.BlockSpec(memory_space=pl.ANY),
        ],
        out_specs=pl.BlockSpec((NPAIR, B, 128), lambda i: (0, 0, TAIL0 // 128)),
        out_shape=jax.ShapeDtypeStruct((NPAIR, B, OUTW), jnp.float32),
        input_output_aliases={5: 0},
    )(pred_emo, pos_emb_weight, emo_emb_weight, jnp.asarray(SEL_E),
      jnp.asarray(SELR_COEFF), big)

    couples = jnp.transpose(big, (1, 0, 2))
    return (couples, jnp.asarray(EMO_CAU))
